# Initial kernel scaffold; baseline (speedup 1.0000x reference)
import jax
import jax.numpy as jnp
from jax.experimental import pallas as pl


def kernel(H0, C0, points0, points1, contents1, motions1, W_I, b_I, W_F, b_F, W_O, b_O, W_C0, b_C0, W_C1, b_C1):
    raise NotImplementedError("write your pallas kernel here")


# speedup vs baseline: 10.0725x; 10.0725x over previous
import functools

import jax
import jax.numpy as jnp
from jax import lax
from jax.experimental import pallas as pl
from jax.experimental.pallas import tpu as pltpu, tpu_sc as plsc

_NHEAD = 5
_CH = 64 * _NHEAD
_CHP = 384
_K = 16


def _a_body(h0_ref, c0_ref, p0_ref, wht_ref, wct_ref, wgx_ref, out_ref):
    p = jnp.dot(p0_ref[...], wgx_ref[...], preferred_element_type=jnp.float32)
    th = jnp.dot(h0_ref[...], wht_ref[...], preferred_element_type=jnp.float32)
    tc = jnp.dot(c0_ref[...], wct_ref[...], preferred_element_type=jnp.float32)
    ta = th.shape[0]
    out_ref[...] = jnp.concatenate(
        [th, tc + p[:, 256:], jnp.zeros((ta, _CHP - _CH), jnp.float32)],
        axis=1) + jnp.concatenate(
        [p[:, :256], jnp.zeros((ta, _CHP - 256), jnp.float32)], axis=1)


def _dense_transform(h0f, c0f, p0f, wht, wct, wgx, interpret=False):
    bn = h0f.shape[0]
    ta = 1024 if bn % 1024 == 0 else bn
    grid = (bn // ta,)
    return pl.pallas_call(
        _a_body,
        grid=grid,
        in_specs=[
            pl.BlockSpec((ta, 64), lambda i: (i, 0)),
            pl.BlockSpec((ta, 64), lambda i: (i, 0)),
            pl.BlockSpec((ta, 8), lambda i: (i, 0)),
            pl.BlockSpec((64, 256), lambda i: (0, 0)),
            pl.BlockSpec((64, 64), lambda i: (0, 0)),
            pl.BlockSpec((8, _CH), lambda i: (0, 0)),
        ],
        out_specs=pl.BlockSpec((ta, _CHP), lambda i: (i, 0)),
        out_shape=jax.ShapeDtypeStruct((bn, _CHP), jnp.float32),
        interpret=interpret,
    )(h0f, c0f, p0f, wht, wct, wgx)


def _b_body(p1_ref, p0_ref, idx_ref, dist_ref, *, n):
    b = pl.program_id(0)
    p1 = p1_ref[0]
    p0 = p0_ref[0]
    dots = jnp.dot(p1, p0, preferred_element_type=jnp.float32)
    p1n = jnp.sum(p1 * p1, axis=1)
    p0n = jnp.sum(p0 * p0, axis=0)
    d0 = p1n[:, None] - 2.0 * dots + p0n[None, :]
    col = lax.broadcasted_iota(jnp.int32, d0.shape, 1)
    tm = d0.shape[0]
    klane = lax.broadcasted_iota(jnp.int32, (tm, _K), 1)

    def round_fn(t, carry):
        d, oidx, odist = carry
        m = jnp.min(d, axis=1)
        sel = jnp.where(d == m[:, None], col, jnp.int32(n))
        it = jnp.min(sel, axis=1)
        dist_t = jnp.sqrt(jnp.maximum(m, 0.0))
        oidx = jnp.where(klane == t, it[:, None], oidx)
        odist = jnp.where(klane == t, dist_t[:, None], odist)
        d = jnp.where(col == it[:, None], jnp.float32(jnp.inf), d)
        return d, oidx, odist

    _, oidx, odist = lax.fori_loop(
        0, _K, round_fn,
        (d0, jnp.zeros((tm, _K), jnp.int32), jnp.zeros((tm, _K), jnp.float32)))
    idx_ref[0] = oidx + b * jnp.int32(n)
    dist_ref[0] = odist


def _knn(p1t, p0t, interpret=False):
    bb, n, _ = p1t.shape
    tm = 256 if n % 256 == 0 else n
    grid = (bb, n // tm)
    return pl.pallas_call(
        functools.partial(_b_body, n=n),
        grid=grid,
        in_specs=[
            pl.BlockSpec((1, tm, 8), lambda b, i: (b, i, 0)),
            pl.BlockSpec((1, 8, n), lambda b, i: (b, 0, 0)),
        ],
        out_specs=[
            pl.BlockSpec((1, tm, _K), lambda b, i: (b, i, 0)),
            pl.BlockSpec((1, tm, _K), lambda b, i: (b, i, 0)),
        ],
        out_shape=[
            jax.ShapeDtypeStruct((bb, n, _K), jnp.int32),
            jax.ShapeDtypeStruct((bb, n, _K), jnp.float32),
        ],
        interpret=interpret,
    )(p1t, p0t)


_NC, _NS = 2, 16
_NW = _NC * _NS
_CA = 8
_NR = _CA * _K
_NV = _CH // 16


def _sc_gather_max(t2, idxf, distf, wgd):
    bn = t2.shape[0]
    apw = bn // _NW
    chunks = apw // _CA

    @functools.partial(
        pl.kernel,
        mesh=plsc.VectorSubcoreMesh(core_axis_name="c", subcore_axis_name="s"),
        out_type=jax.ShapeDtypeStruct((bn, _CH), jnp.float32),
        scratch_types=[
            pltpu.VMEM((_NR,), jnp.int32),
            pltpu.VMEM((_NR,), jnp.float32),
            pltpu.VMEM((_NR, _CHP), jnp.float32),
            pltpu.VMEM((_CA, _CH), jnp.float32),
            pltpu.VMEM((_CH,), jnp.float32),
            pltpu.SemaphoreType.DMA,
        ],
    )
    def sck(t2_hbm, idx_hbm, dist_hbm, wgd_hbm, out_hbm,
            idx_v, dist_v, rows_v, out_v, wgd_v, sem):
        wid = lax.axis_index("s") * _NC + lax.axis_index("c")
        pltpu.sync_copy(wgd_hbm, wgd_v)

        def chunk(ci, carry):
            a0 = wid * apw + ci * _CA
            off = a0 * _K
            pltpu.sync_copy(idx_hbm.at[pl.ds(off, _NR)], idx_v)
            pltpu.sync_copy(dist_hbm.at[pl.ds(off, _NR)], dist_v)
            pltpu.async_copy(t2_hbm.at[idx_v], rows_v, sem).wait()

            def anchor(a, c2):
                accs = [jnp.full((16,), -3.4e38, jnp.float32)
                        for _ in range(_NV)]
                dvec = dist_v[pl.ds(a * _K, _K)]
                for k in range(_K):
                    r = a * _K + k
                    dspl = dvec[k]
                    for c in range(_NV):
                        v = rows_v[r, pl.ds(c * 16, 16)] \
                            + dspl * wgd_v[pl.ds(c * 16, 16)]
                        accs[c] = jnp.maximum(accs[c], v)
                for c in range(_NV):
                    out_v[a, pl.ds(c * 16, 16)] = accs[c]
                return c2

            lax.fori_loop(0, _CA, anchor, 0)
            pltpu.sync_copy(out_v, out_hbm.at[pl.ds(a0, _CA)])
            return carry

        lax.fori_loop(0, chunks, chunk, 0)

    return sck(t2, idxf, distf, wgd)


def _d_body(m_ref, f_ref, p1_ref, wft_ref, wgx_ref, b_ref, h1_ref, c1_ref):
    u = jnp.dot(f_ref[...], wft_ref[...], preferred_element_type=jnp.float32)
    s = jnp.dot(p1_ref[...], wgx_ref[...], preferred_element_type=jnp.float32)
    td = u.shape[0]
    pre = m_ref[...] - s + b_ref[...]
    pre = pre + jnp.concatenate(
        [u, jnp.zeros((td, 64), jnp.float32)], axis=1)
    g_i = jax.nn.sigmoid(pre[:, 0:64])
    g_f = jax.nn.sigmoid(pre[:, 64:128])
    g_o = jax.nn.sigmoid(pre[:, 128:192])
    c1a = jnp.tanh(pre[:, 192:256])
    c0a = pre[:, 256:320]
    c1 = g_f * c0a + g_i * c1a
    h1_ref[...] = g_o * jnp.tanh(c1)
    c1_ref[...] = c1


def _combine(m, featsf, p1f, wft, wgx, bvec, interpret=False):
    bn = m.shape[0]
    td = 1024 if bn % 1024 == 0 else bn
    grid = (bn // td,)
    return pl.pallas_call(
        _d_body,
        grid=grid,
        in_specs=[
            pl.BlockSpec((td, _CH), lambda i: (i, 0)),
            pl.BlockSpec((td, 128), lambda i: (i, 0)),
            pl.BlockSpec((td, 8), lambda i: (i, 0)),
            pl.BlockSpec((128, 256), lambda i: (0, 0)),
            pl.BlockSpec((8, _CH), lambda i: (0, 0)),
            pl.BlockSpec((1, _CH), lambda i: (0, 0)),
        ],
        out_specs=[
            pl.BlockSpec((td, 64), lambda i: (i, 0)),
            pl.BlockSpec((td, 64), lambda i: (i, 0)),
        ],
        out_shape=[
            jax.ShapeDtypeStruct((bn, 64), jnp.float32),
            jax.ShapeDtypeStruct((bn, 64), jnp.float32),
        ],
        interpret=interpret,
    )(m, featsf, p1f, wft, wgx, bvec)


def kernel(H0, C0, points0, points1, contents1, motions1,
           W_I, b_I, W_F, b_F, W_O, b_O, W_C0, b_C0, W_C1, b_C1):
    bb, hid, n = H0.shape
    bn = bb * n

    wg = jnp.concatenate(
        [W_I[:, :4], W_F[:, :4], W_O[:, :4], W_C1[:, :4], W_C0[:, :4]], axis=0)
    wgx = jnp.concatenate(
        [wg[:, :3].T, jnp.zeros((5, _CH), jnp.float32)], axis=0)
    wgd = wg[:, 3]
    wht = jnp.concatenate(
        [W_I[:, 4:68], W_F[:, 4:68], W_O[:, 4:68], W_C1[:, 4:68]],
        axis=0).T
    wct = W_C0[:, 4:68].T
    wft = jnp.concatenate(
        [W_I[:, 68:], W_F[:, 68:], W_O[:, 68:], W_C1[:, 68:]],
        axis=0).T
    bvec = jnp.concatenate([b_I, b_F, b_O, b_C1, b_C0]).reshape(1, _CH)

    h0f = H0.transpose(0, 2, 1).reshape(bn, hid)
    c0f = C0.transpose(0, 2, 1).reshape(bn, hid)
    pad5 = jnp.zeros((bb, 5, n), jnp.float32)
    p0t = jnp.concatenate([points0, pad5], axis=1)
    p0f = p0t.transpose(0, 2, 1).reshape(bn, 8)
    p1t = jnp.concatenate([points1, pad5], axis=1).transpose(0, 2, 1)
    p1f = p1t.reshape(bn, 8)
    featsf = jnp.concatenate([contents1, motions1], axis=1) \
        .transpose(0, 2, 1).reshape(bn, 128)

    t2 = _dense_transform(h0f, c0f, p0f, wht, wct, wgx)
    idx, dist = _knn(p1t, p0t)
    m = _sc_gather_max(t2, idx.reshape(-1), dist.reshape(-1), wgd)
    h1f, c1f = _combine(m, featsf, p1f, wft, wgx, bvec)

    h1 = h1f.reshape(bb, n, hid).transpose(0, 2, 1)
    c1 = c1f.reshape(bb, n, hid).transpose(0, 2, 1)
    return (h1, c1)
